# baseline (speedup 1.0000x reference)
import functools

import jax
import jax.numpy as jnp
from jax import lax
from jax.experimental import pallas as pl
from jax.experimental.pallas import tpu as pltpu
from jax.experimental.pallas import tpu_sc as plsc

_DIM = 64
_CTX = 5
_LANES = 16


def _sc_word2vec(B, ctx, dim):
    NW = 32
    b_per_w = B // NW
    CHUNK = 128
    n_chunks = b_per_w // CHUNK
    CB = CHUNK * ctx

    mesh = plsc.VectorSubcoreMesh(core_axis_name="c", subcore_axis_name="s")

    @functools.partial(
        pl.kernel,
        out_type=jax.ShapeDtypeStruct((B * ctx,), jnp.float32),
        mesh=mesh,
        scratch_types=[
            pltpu.VMEM((CHUNK,), jnp.int32),
            pltpu.VMEM((CB,), jnp.int32),
            pltpu.VMEM((CHUNK, dim), jnp.float32),
            pltpu.VMEM((CB, dim), jnp.float32),
            pltpu.VMEM((CB,), jnp.float32),
            pltpu.SemaphoreType.DMA,
            pltpu.SemaphoreType.DMA,
        ],
        compiler_params=pltpu.CompilerParams(
            needs_layout_passes=False, use_tc_tiling_on_sc=False),
    )
    def k(tgt_hbm, ctxi_hbm, ttab_hbm, ctab_hbm, out_hbm,
          idx_t, idx_c, trows, crows, outv, sem_t, sem_c):
        wid = lax.axis_index("s") * 2 + lax.axis_index("c")
        base_b = wid * b_per_w
        lane = lax.iota(jnp.int32, _LANES)

        for ci in range(n_chunks):
            off_b = pl.multiple_of(base_b + ci * CHUNK, CHUNK)
            off_c = pl.multiple_of((base_b + ci * CHUNK) * ctx, CB)
            pltpu.sync_copy(tgt_hbm.at[pl.ds(off_b, CHUNK)], idx_t)
            pltpu.sync_copy(ctxi_hbm.at[pl.ds(off_c, CB)], idx_c)
            cp_t = pltpu.async_copy(ttab_hbm.at[idx_t], trows, sem_t)
            cps = []
            for j in range(ctx):
                cps.append(pltpu.async_copy(
                    ctab_hbm.at[idx_c.at[pl.ds(j * CHUNK, CHUNK)]],
                    crows.at[pl.ds(j * CHUNK, CHUNK), :], sem_c))
            cp_t.wait()
            for cp in cps:
                cp.wait()

            lane_masks = [lane == j for j in range(_LANES)]

            def q_body(q, carry):
                res = jnp.zeros((_LANES,), jnp.float32)
                for j in range(_LANES):
                    p = q * _LANES + j
                    b = p // ctx
                    c = p - b * ctx
                    s = jnp.zeros((_LANES,), jnp.float32)
                    for k in range(dim // _LANES):
                        sl = pl.ds(k * _LANES, _LANES)
                        s = s + trows[b, sl] * crows[b * ctx + c, sl]
                    res = jnp.where(lane_masks[j], jnp.sum(s), res)
                outv[pl.ds(q * _LANES, _LANES)] = res
                return carry

            lax.fori_loop(0, CB // _LANES, q_body, 0)
            pltpu.sync_copy(outv, out_hbm.at[pl.ds(off_c, CB)])

    return k


def kernel(target, context, target_table, context_table):
    B, ctx = context.shape
    dim = target_table.shape[1]
    tgt = target.reshape(B).astype(jnp.int32)
    ctxi = context.reshape(B * ctx).astype(jnp.int32)
    out = _sc_word2vec(B, ctx, dim)(tgt, ctxi, target_table, context_table)
    return out.reshape(B, ctx)
